# baseline (speedup 1.0000x reference)
import functools

import jax
import jax.numpy as jnp
from jax import lax
from jax.experimental import pallas as pl
from jax.experimental.pallas import tpu as pltpu

OBS_DIM = 512
HIDDEN = 2048
NUM_ACTIONS = 32768
BATCH = 64
TOP_K = 64
MIN_PROB = 1e-4

BN = 2048
NBLK = NUM_ACTIONS // BN
NEG = -1e30


def _mlp_stats_kernel(s_ref, W1_ref, b1_ref, W2_ref, b2_ref, legal_ref,
                      key_ref, mb_ref, lsb_ref, h_ref, m_ref, ssum_ref):
    j = pl.program_id(0)

    @pl.when(j == 0)
    def _():
        h_ref[...] = jnp.tanh(
            jnp.dot(s_ref[...], W1_ref[...],
                    preferred_element_type=jnp.float32) + b1_ref[...])

    x = jnp.dot(h_ref[...], W2_ref[...],
                preferred_element_type=jnp.float32) + b2_ref[...]
    bmax = jnp.max(x, axis=1, keepdims=True)
    key_ref[...] = jnp.where(legal_ref[...] != 0.0, x, NEG)

    @pl.when(j == 0)
    def _():
        m_ref[...] = bmax
        ssum_ref[...] = jnp.sum(jnp.exp(x - bmax), axis=1, keepdims=True)

    @pl.when(j != 0)
    def _():
        m_old = m_ref[...]
        m_new = jnp.maximum(m_old, bmax)
        ssum_ref[...] = (ssum_ref[...] * jnp.exp(m_old - m_new)
                         + jnp.sum(jnp.exp(x - m_new), axis=1, keepdims=True))
        m_ref[...] = m_new

    @pl.when(j == NBLK - 1)
    def _():
        mb_ref[...] = jnp.broadcast_to(m_ref[...], (BATCH, 16))
        lsb_ref[...] = jnp.broadcast_to(jnp.log(ssum_ref[...]), (BATCH, 16))


def _probs_kernel(key_ref, mb_ref, lsb_ref, p_ref):
    t = (key_ref[...] - mb_ref[:, :1]) - lsb_ref[:, :1]
    p_ref[...] = jnp.exp(t)


def _mlp_probs(s, legal_actions, W1, b1, W2, b2):
    b1r = b1.reshape(1, HIDDEN)
    b2r = b2.reshape(1, NUM_ACTIONS)
    key, mb, lsb = pl.pallas_call(
        _mlp_stats_kernel,
        grid=(NBLK,),
        in_specs=[
            pl.BlockSpec((BATCH, OBS_DIM), lambda j: (0, 0)),
            pl.BlockSpec((OBS_DIM, HIDDEN), lambda j: (0, 0)),
            pl.BlockSpec((1, HIDDEN), lambda j: (0, 0)),
            pl.BlockSpec((HIDDEN, BN), lambda j: (0, j)),
            pl.BlockSpec((1, BN), lambda j: (0, j)),
            pl.BlockSpec((BATCH, BN), lambda j: (0, j)),
        ],
        out_specs=[
            pl.BlockSpec((BATCH, BN), lambda j: (0, j)),
            pl.BlockSpec((BATCH, 16), lambda j: (0, 0)),
            pl.BlockSpec((BATCH, 16), lambda j: (0, 0)),
        ],
        out_shape=[
            jax.ShapeDtypeStruct((BATCH, NUM_ACTIONS), jnp.float32),
            jax.ShapeDtypeStruct((BATCH, 16), jnp.float32),
            jax.ShapeDtypeStruct((BATCH, 16), jnp.float32),
        ],
        scratch_shapes=[
            pltpu.VMEM((BATCH, HIDDEN), jnp.float32),
            pltpu.VMEM((BATCH, 1), jnp.float32),
            pltpu.VMEM((BATCH, 1), jnp.float32),
        ],
        compiler_params=pltpu.CompilerParams(
            dimension_semantics=("arbitrary",)),
    )(s, W1, b1r, W2, b2r, legal_actions)

    p = pl.pallas_call(
        _probs_kernel,
        grid=(NBLK,),
        in_specs=[
            pl.BlockSpec((BATCH, BN), lambda j: (0, j)),
            pl.BlockSpec((BATCH, 16), lambda j: (0, 0)),
            pl.BlockSpec((BATCH, 16), lambda j: (0, 0)),
        ],
        out_specs=pl.BlockSpec((BATCH, BN), lambda j: (0, j)),
        out_shape=jax.ShapeDtypeStruct((BATCH, NUM_ACTIONS), jnp.float32),
    )(key, mb, lsb)
    return p


def kernel(s, legal_actions, W1, b1, W2, b2, top_k):
    p = _mlp_probs(s, legal_actions, W1, b1, W2, b2)
    vals, idx = lax.top_k(p, TOP_K)
    valid = jnp.arange(TOP_K) < top_k
    mask = (vals > MIN_PROB) & valid
    top_k_actions = jnp.where(mask, idx, -1).astype(jnp.int32)
    top_k_probs = jnp.where(mask, vals, 0.0)
    return top_k_actions, top_k_probs


# speedup vs baseline: 2.7934x; 2.7934x over previous
import functools

import jax
import jax.numpy as jnp
from jax import lax
from jax.experimental import pallas as pl
from jax.experimental.pallas import tpu as pltpu
from jax.experimental.pallas import tpu_sc as plsc

OBS_DIM = 512
HIDDEN = 2048
NUM_ACTIONS = 32768
BATCH = 64
TOP_K = 64
MIN_PROB = 1e-4

BN = 2048
NBLK = NUM_ACTIONS // BN
NEG = -1e30


def _mlp_stats_kernel(s_ref, W1_ref, b1_ref, W2_ref, b2_ref, legal_ref,
                      key_ref, mb_ref, lsb_ref, h_ref, m_ref, ssum_ref):
    j = pl.program_id(0)

    @pl.when(j == 0)
    def _():
        h_ref[...] = jnp.tanh(
            jnp.dot(s_ref[...], W1_ref[...],
                    preferred_element_type=jnp.float32) + b1_ref[...])

    x = jnp.dot(h_ref[...], W2_ref[...],
                preferred_element_type=jnp.float32) + b2_ref[...]
    bmax = jnp.max(x, axis=1, keepdims=True)
    key_ref[...] = jnp.where(legal_ref[...] != 0.0, x, NEG)

    @pl.when(j == 0)
    def _():
        m_ref[...] = bmax
        ssum_ref[...] = jnp.sum(jnp.exp(x - bmax), axis=1, keepdims=True)

    @pl.when(j != 0)
    def _():
        m_old = m_ref[...]
        m_new = jnp.maximum(m_old, bmax)
        ssum_ref[...] = (ssum_ref[...] * jnp.exp(m_old - m_new)
                         + jnp.sum(jnp.exp(x - m_new), axis=1, keepdims=True))
        m_ref[...] = m_new

    @pl.when(j == NBLK - 1)
    def _():
        mb_ref[...] = jnp.broadcast_to(m_ref[...], (BATCH, 16))
        lsb_ref[...] = jnp.broadcast_to(jnp.log(ssum_ref[...]), (BATCH, 16))


def _probs_kernel(key_ref, mb_ref, lsb_ref, p_ref):
    t = (key_ref[...] - mb_ref[:, :1]) - lsb_ref[:, :1]
    p_ref[...] = jnp.exp(t)


def _mlp_probs(s, legal_actions, W1, b1, W2, b2):
    b1r = b1.reshape(1, HIDDEN)
    b2r = b2.reshape(1, NUM_ACTIONS)
    key, mb, lsb = pl.pallas_call(
        _mlp_stats_kernel,
        grid=(NBLK,),
        in_specs=[
            pl.BlockSpec((BATCH, OBS_DIM), lambda j: (0, 0)),
            pl.BlockSpec((OBS_DIM, HIDDEN), lambda j: (0, 0)),
            pl.BlockSpec((1, HIDDEN), lambda j: (0, 0)),
            pl.BlockSpec((HIDDEN, BN), lambda j: (0, j)),
            pl.BlockSpec((1, BN), lambda j: (0, j)),
            pl.BlockSpec((BATCH, BN), lambda j: (0, j)),
        ],
        out_specs=[
            pl.BlockSpec((BATCH, BN), lambda j: (0, j)),
            pl.BlockSpec((BATCH, 16), lambda j: (0, 0)),
            pl.BlockSpec((BATCH, 16), lambda j: (0, 0)),
        ],
        out_shape=[
            jax.ShapeDtypeStruct((BATCH, NUM_ACTIONS), jnp.float32),
            jax.ShapeDtypeStruct((BATCH, 16), jnp.float32),
            jax.ShapeDtypeStruct((BATCH, 16), jnp.float32),
        ],
        scratch_shapes=[
            pltpu.VMEM((BATCH, HIDDEN), jnp.float32),
            pltpu.VMEM((BATCH, 1), jnp.float32),
            pltpu.VMEM((BATCH, 1), jnp.float32),
        ],
        compiler_params=pltpu.CompilerParams(
            dimension_semantics=("arbitrary",)),
    )(s, W1, b1r, W2, b2r, legal_actions)

    p = pl.pallas_call(
        _probs_kernel,
        grid=(NBLK,),
        in_specs=[
            pl.BlockSpec((BATCH, BN), lambda j: (0, j)),
            pl.BlockSpec((BATCH, 16), lambda j: (0, 0)),
            pl.BlockSpec((BATCH, 16), lambda j: (0, 0)),
        ],
        out_specs=pl.BlockSpec((BATCH, BN), lambda j: (0, j)),
        out_shape=jax.ShapeDtypeStruct((BATCH, NUM_ACTIONS), jnp.float32),
    )(key, mb, lsb)
    return p


CAP = 10240
L = 16
NW = 32
ROWS_PER_W = BATCH // NW


def _cmp_wins(ak, ai, bk, bi):
    return (ak > bk) | ((ak == bk) & (ai < bi))


def _exchange(lo_k, lo_i, hi_k, hi_i):
    rb_k = lax.rev(hi_k, (0,))
    rb_i = lax.rev(hi_i, (0,))
    w = _cmp_wins(lo_k, lo_i, rb_k, rb_i)
    top_k_ = jnp.where(w, lo_k, rb_k)
    top_i_ = jnp.where(w, lo_i, rb_i)
    bot_k_ = jnp.where(w, rb_k, lo_k)
    bot_i_ = jnp.where(w, rb_i, lo_i)
    bk2, bi2 = plsc.sort_key_val(bot_k_, bot_i_)
    tk2, ti2 = plsc.sort_key_val(top_k_, top_i_)
    return bk2, bi2, tk2, ti2


def _topk_sc_kernel(p_hbm, act_hbm, prob_hbm, row_v, cand_v, candi_v,
                    acto_v, probo_v):
    cid = lax.axis_index("c")
    sid = lax.axis_index("s")
    wid = sid * 2 + cid

    for rr in range(ROWS_PER_W):
        r = wid * ROWS_PER_W + rr
        pltpu.sync_copy(p_hbm.at[r], row_v)

        lane = lax.iota(jnp.int32, L)

        def compact_body(i, off):
            v = row_v[pl.ds(i * L, L)]
            m = v > MIN_PROB
            sk, si = plsc.sort_key_val(jnp.where(m, v, -1.0), lane + i * L,
                                       descending=True)
            cand_v[pl.ds(off, L)] = sk
            candi_v[pl.ds(off, L)] = si
            return off + jnp.sum(m.astype(jnp.int32))

        off = lax.fori_loop(0, NUM_ACTIONS // L, compact_body, 0)

        cand_v[pl.ds(off, L)] = jnp.full((L,), -1.0, jnp.float32)
        candi_v[pl.ds(off, L)] = jnp.full((L,), -1, jnp.int32)
        nv = off // L + 1

        init = (jnp.full((L,), -1.0, jnp.float32), jnp.full((L,), -1, jnp.int32),
                jnp.full((L,), -1.0, jnp.float32), jnp.full((L,), -1, jnp.int32),
                jnp.full((L,), -1.0, jnp.float32), jnp.full((L,), -1, jnp.int32),
                jnp.full((L,), -1.0, jnp.float32), jnp.full((L,), -1, jnp.int32),
                jnp.float32(-1.0))

        def merge_body(k, carry):
            t0k, t0i, t1k, t1i, t2k, t2i, t3k, t3i, tmin = carry
            w = cand_v[pl.ds(k * L, L)]
            wi = candi_v[pl.ds(k * L, L)]
            wmax = jnp.max(w)

            def do_merge(_):
                ws, wis = plsc.sort_key_val(w, wi)
                rb_k = lax.rev(ws, (0,))
                rb_i = lax.rev(wis, (0,))
                win = _cmp_wins(t0k, t0i, rb_k, rb_i)
                n0k = jnp.where(win, t0k, rb_k)
                n0i = jnp.where(win, t0i, rb_i)
                n0k, n0i = plsc.sort_key_val(n0k, n0i)
                a0k, a0i, a1k, a1i = _exchange(n0k, n0i, t1k, t1i)
                a1k, a1i, a2k, a2i = _exchange(a1k, a1i, t2k, t2i)
                a2k, a2i, a3k, a3i = _exchange(a2k, a2i, t3k, t3i)
                return (a0k, a0i, a1k, a1i, a2k, a2i, a3k, a3i,
                        jnp.min(a0k))

            return lax.cond(wmax >= tmin, do_merge, lambda _: carry, None)

        t0k, t0i, t1k, t1i, t2k, t2i, t3k, t3i, _ = lax.fori_loop(
            0, nv, merge_body, init)

        for q, (tk_, ti_) in enumerate(
                ((t3k, t3i), (t2k, t2i), (t1k, t1i), (t0k, t0i))):
            m = tk_ > MIN_PROB
            a = jnp.where(m, ti_, -1)
            pv = jnp.where(m, tk_, 0.0)
            acto_v[pl.ds(q * L, L)] = lax.rev(a, (0,))
            probo_v[pl.ds(q * L, L)] = lax.rev(pv, (0,))

        pltpu.sync_copy(acto_v, act_hbm.at[r])
        pltpu.sync_copy(probo_v, prob_hbm.at[r])


def _topk_sc(p):
    mesh = plsc.VectorSubcoreMesh(core_axis_name="c", subcore_axis_name="s")
    f = pl.kernel(
        _topk_sc_kernel,
        out_type=(jax.ShapeDtypeStruct((BATCH, TOP_K), jnp.int32),
                  jax.ShapeDtypeStruct((BATCH, TOP_K), jnp.float32)),
        mesh=mesh,
        scratch_types=[
            pltpu.VMEM((NUM_ACTIONS,), jnp.float32),
            pltpu.VMEM((CAP,), jnp.float32),
            pltpu.VMEM((CAP,), jnp.int32),
            pltpu.VMEM((TOP_K,), jnp.int32),
            pltpu.VMEM((TOP_K,), jnp.float32),
        ],
        compiler_params=pltpu.CompilerParams(needs_layout_passes=False),
    )
    return f(p)


def kernel(s, legal_actions, W1, b1, W2, b2, top_k):
    p = _mlp_probs(s, legal_actions, W1, b1, W2, b2)
    acts, probs = _topk_sc(p)
    valid = jnp.arange(TOP_K) < top_k
    top_k_actions = jnp.where(valid, acts, -1).astype(jnp.int32)
    top_k_probs = jnp.where(valid, probs, 0.0)
    return top_k_actions, top_k_probs


# speedup vs baseline: 2.8762x; 1.0296x over previous
import functools

import jax
import jax.numpy as jnp
from jax import lax
from jax.experimental import pallas as pl
from jax.experimental.pallas import tpu as pltpu
from jax.experimental.pallas import tpu_sc as plsc

OBS_DIM = 512
HIDDEN = 2048
NUM_ACTIONS = 32768
BATCH = 64
TOP_K = 64
MIN_PROB = 1e-4

BN = 2048
NBLK = NUM_ACTIONS // BN
NEG = -1e30


def _mlp_stats_kernel(s_ref, W1_ref, b1_ref, W2_ref, b2_ref, legal_ref,
                      key_ref, mb_ref, lsb_ref, h_ref, m_ref, ssum_ref):
    j = pl.program_id(0)

    @pl.when(j == 0)
    def _():
        h_ref[...] = jnp.tanh(
            jnp.dot(s_ref[...], W1_ref[...],
                    preferred_element_type=jnp.float32) + b1_ref[...])

    x = jnp.dot(h_ref[...], W2_ref[...],
                preferred_element_type=jnp.float32) + b2_ref[...]
    bmax = jnp.max(x, axis=1, keepdims=True)
    key_ref[...] = jnp.where(legal_ref[...] != 0.0, x, NEG)

    @pl.when(j == 0)
    def _():
        m_ref[...] = bmax
        ssum_ref[...] = jnp.sum(jnp.exp(x - bmax), axis=1, keepdims=True)

    @pl.when(j != 0)
    def _():
        m_old = m_ref[...]
        m_new = jnp.maximum(m_old, bmax)
        ssum_ref[...] = (ssum_ref[...] * jnp.exp(m_old - m_new)
                         + jnp.sum(jnp.exp(x - m_new), axis=1, keepdims=True))
        m_ref[...] = m_new

    @pl.when(j == NBLK - 1)
    def _():
        mb_ref[...] = jnp.broadcast_to(m_ref[...], (BATCH, 16))
        lsb_ref[...] = jnp.broadcast_to(jnp.log(ssum_ref[...]), (BATCH, 16))


def _probs_kernel(key_ref, mb_ref, lsb_ref, p_ref):
    t = (key_ref[...] - mb_ref[:, :1]) - lsb_ref[:, :1]
    p_ref[...] = jnp.exp(t)


def _mlp_probs(s, legal_actions, W1, b1, W2, b2):
    b1r = b1.reshape(1, HIDDEN)
    b2r = b2.reshape(1, NUM_ACTIONS)
    key, mb, lsb = pl.pallas_call(
        _mlp_stats_kernel,
        grid=(NBLK,),
        in_specs=[
            pl.BlockSpec((BATCH, OBS_DIM), lambda j: (0, 0)),
            pl.BlockSpec((OBS_DIM, HIDDEN), lambda j: (0, 0)),
            pl.BlockSpec((1, HIDDEN), lambda j: (0, 0)),
            pl.BlockSpec((HIDDEN, BN), lambda j: (0, j)),
            pl.BlockSpec((1, BN), lambda j: (0, j)),
            pl.BlockSpec((BATCH, BN), lambda j: (0, j)),
        ],
        out_specs=[
            pl.BlockSpec((BATCH, BN), lambda j: (0, j)),
            pl.BlockSpec((BATCH, 16), lambda j: (0, 0)),
            pl.BlockSpec((BATCH, 16), lambda j: (0, 0)),
        ],
        out_shape=[
            jax.ShapeDtypeStruct((BATCH, NUM_ACTIONS), jnp.float32),
            jax.ShapeDtypeStruct((BATCH, 16), jnp.float32),
            jax.ShapeDtypeStruct((BATCH, 16), jnp.float32),
        ],
        scratch_shapes=[
            pltpu.VMEM((BATCH, HIDDEN), jnp.float32),
            pltpu.VMEM((BATCH, 1), jnp.float32),
            pltpu.VMEM((BATCH, 1), jnp.float32),
        ],
        compiler_params=pltpu.CompilerParams(
            dimension_semantics=("arbitrary",)),
    )(s, W1, b1r, W2, b2r, legal_actions)

    p = pl.pallas_call(
        _probs_kernel,
        grid=(NBLK,),
        in_specs=[
            pl.BlockSpec((BATCH, BN), lambda j: (0, j)),
            pl.BlockSpec((BATCH, 16), lambda j: (0, 0)),
            pl.BlockSpec((BATCH, 16), lambda j: (0, 0)),
        ],
        out_specs=pl.BlockSpec((BATCH, BN), lambda j: (0, j)),
        out_shape=jax.ShapeDtypeStruct((BATCH, NUM_ACTIONS), jnp.float32),
    )(key, mb, lsb)
    return p


CAP = 10240
L = 16
NW = 32
ROWS_PER_W = BATCH // NW


def _cmp_wins(ak, ai, bk, bi):
    return (ak > bk) | ((ak == bk) & (ai < bi))


def _exchange(lo_k, lo_i, hi_k, hi_i):
    rb_k = lax.rev(hi_k, (0,))
    rb_i = lax.rev(hi_i, (0,))
    w = _cmp_wins(lo_k, lo_i, rb_k, rb_i)
    top_k_ = jnp.where(w, lo_k, rb_k)
    top_i_ = jnp.where(w, lo_i, rb_i)
    bot_k_ = jnp.where(w, rb_k, lo_k)
    bot_i_ = jnp.where(w, rb_i, lo_i)
    bk2, bi2 = plsc.sort_key_val(bot_k_, bot_i_)
    tk2, ti2 = plsc.sort_key_val(top_k_, top_i_)
    return bk2, bi2, tk2, ti2


def _topk_sc_kernel(p_hbm, act_hbm, prob_hbm, row_a, row_b, cand_v, candi_v,
                    acto_v, probo_v, sem_a, sem_b):
    cid = lax.axis_index("c")
    sid = lax.axis_index("s")
    wid = sid * 2 + cid

    cp_a = pltpu.async_copy(p_hbm.at[wid * ROWS_PER_W], row_a, sem_a)
    cp_b = pltpu.async_copy(p_hbm.at[wid * ROWS_PER_W + 1], row_b, sem_b)

    lane = lax.iota(jnp.int32, L)

    for rr, (row_v, cp) in enumerate(((row_a, cp_a), (row_b, cp_b))):
        r = wid * ROWS_PER_W + rr
        cp.wait()

        def compact_body(i, off):
            v = row_v[pl.ds(i * L, L)]
            m = v > MIN_PROB
            mi = m.astype(jnp.int32)
            cs = plsc.cumsum(mi)
            pos = jnp.where(m, off + cs - 1, CAP + lane)
            plsc.store_scatter(cand_v, [pos], v)
            plsc.store_scatter(candi_v, [pos], lane + i * L)
            return off + jnp.sum(mi)

        off = lax.fori_loop(0, NUM_ACTIONS // L, compact_body, 0)

        cand_v[pl.ds(off, L)] = jnp.full((L,), -1.0, jnp.float32)
        candi_v[pl.ds(off, L)] = jnp.full((L,), -1, jnp.int32)
        nv = off // L + 1

        init = (jnp.full((L,), -1.0, jnp.float32), jnp.full((L,), -1, jnp.int32),
                jnp.full((L,), -1.0, jnp.float32), jnp.full((L,), -1, jnp.int32),
                jnp.full((L,), -1.0, jnp.float32), jnp.full((L,), -1, jnp.int32),
                jnp.full((L,), -1.0, jnp.float32), jnp.full((L,), -1, jnp.int32),
                jnp.float32(-1.0))

        def merge_body(k, carry):
            t0k, t0i, t1k, t1i, t2k, t2i, t3k, t3i, tmin = carry
            w = cand_v[pl.ds(k * L, L)]
            wi = candi_v[pl.ds(k * L, L)]
            wmax = jnp.max(w)

            def do_merge(_):
                ws, wis = plsc.sort_key_val(w, wi)
                rb_k = lax.rev(ws, (0,))
                rb_i = lax.rev(wis, (0,))
                win = _cmp_wins(t0k, t0i, rb_k, rb_i)
                n0k = jnp.where(win, t0k, rb_k)
                n0i = jnp.where(win, t0i, rb_i)
                n0k, n0i = plsc.sort_key_val(n0k, n0i)
                a0k, a0i, a1k, a1i = _exchange(n0k, n0i, t1k, t1i)
                a1k, a1i, a2k, a2i = _exchange(a1k, a1i, t2k, t2i)
                a2k, a2i, a3k, a3i = _exchange(a2k, a2i, t3k, t3i)
                return (a0k, a0i, a1k, a1i, a2k, a2i, a3k, a3i,
                        jnp.min(a0k))

            return lax.cond(wmax >= tmin, do_merge, lambda _: carry, None)

        t0k, t0i, t1k, t1i, t2k, t2i, t3k, t3i, _ = lax.fori_loop(
            0, nv, merge_body, init)

        for q, (tk_, ti_) in enumerate(
                ((t3k, t3i), (t2k, t2i), (t1k, t1i), (t0k, t0i))):
            m = tk_ > MIN_PROB
            a = jnp.where(m, ti_, -1)
            pv = jnp.where(m, tk_, 0.0)
            acto_v[pl.ds(q * L, L)] = lax.rev(a, (0,))
            probo_v[pl.ds(q * L, L)] = lax.rev(pv, (0,))

        pltpu.sync_copy(acto_v, act_hbm.at[r])
        pltpu.sync_copy(probo_v, prob_hbm.at[r])


def _topk_sc(p):
    mesh = plsc.VectorSubcoreMesh(core_axis_name="c", subcore_axis_name="s")
    f = pl.kernel(
        _topk_sc_kernel,
        out_type=(jax.ShapeDtypeStruct((BATCH, TOP_K), jnp.int32),
                  jax.ShapeDtypeStruct((BATCH, TOP_K), jnp.float32)),
        mesh=mesh,
        scratch_types=[
            pltpu.VMEM((NUM_ACTIONS,), jnp.float32),
            pltpu.VMEM((NUM_ACTIONS,), jnp.float32),
            pltpu.VMEM((CAP + L,), jnp.float32),
            pltpu.VMEM((CAP + L,), jnp.int32),
            pltpu.VMEM((TOP_K,), jnp.int32),
            pltpu.VMEM((TOP_K,), jnp.float32),
            pltpu.SemaphoreType.DMA,
            pltpu.SemaphoreType.DMA,
        ],
        compiler_params=pltpu.CompilerParams(needs_layout_passes=False),
    )
    return f(p)


def kernel(s, legal_actions, W1, b1, W2, b2, top_k):
    p = _mlp_probs(s, legal_actions, W1, b1, W2, b2)
    acts, probs = _topk_sc(p)
    valid = jnp.arange(TOP_K) < top_k
    top_k_actions = jnp.where(valid, acts, -1).astype(jnp.int32)
    top_k_probs = jnp.where(valid, probs, 0.0)
    return top_k_actions, top_k_probs


# speedup vs baseline: 2.8783x; 1.0007x over previous
import functools

import jax
import jax.numpy as jnp
from jax import lax
from jax.experimental import pallas as pl
from jax.experimental.pallas import tpu as pltpu
from jax.experimental.pallas import tpu_sc as plsc

OBS_DIM = 512
HIDDEN = 2048
NUM_ACTIONS = 32768
BATCH = 64
TOP_K = 64
MIN_PROB = 1e-4

BN = 2048
NBLK = NUM_ACTIONS // BN
NEG = -1e30


def _mlp_stats_kernel(s_ref, W1_ref, b1_ref, W2_ref, b2_ref, legal_ref,
                      key_ref, mb_ref, lsb_ref, h_ref, m_ref, ssum_ref):
    j = pl.program_id(0)

    @pl.when(j == 0)
    def _():
        h_ref[...] = jnp.tanh(
            jnp.dot(s_ref[...], W1_ref[...],
                    preferred_element_type=jnp.float32) + b1_ref[...])

    x = jnp.dot(h_ref[...], W2_ref[...],
                preferred_element_type=jnp.float32) + b2_ref[...]
    bmax = jnp.max(x, axis=1, keepdims=True)
    key_ref[...] = jnp.where(legal_ref[...] != 0.0, x, NEG)

    @pl.when(j == 0)
    def _():
        m_ref[...] = bmax
        ssum_ref[...] = jnp.sum(jnp.exp(x - bmax), axis=1, keepdims=True)

    @pl.when(j != 0)
    def _():
        m_old = m_ref[...]
        m_new = jnp.maximum(m_old, bmax)
        ssum_ref[...] = (ssum_ref[...] * jnp.exp(m_old - m_new)
                         + jnp.sum(jnp.exp(x - m_new), axis=1, keepdims=True))
        m_ref[...] = m_new

    @pl.when(j == NBLK - 1)
    def _():
        mb_ref[...] = jnp.broadcast_to(m_ref[...], (BATCH, 16))
        lsb_ref[...] = jnp.broadcast_to(jnp.log(ssum_ref[...]), (BATCH, 16))


def _probs_kernel(key_ref, mb_ref, lsb_ref, p_ref):
    t = (key_ref[...] - mb_ref[:, :1]) - lsb_ref[:, :1]
    p_ref[...] = jnp.exp(t)


def _mlp_probs(s, legal_actions, W1, b1, W2, b2):
    b1r = b1.reshape(1, HIDDEN)
    b2r = b2.reshape(1, NUM_ACTIONS)
    key, mb, lsb = pl.pallas_call(
        _mlp_stats_kernel,
        grid=(NBLK,),
        in_specs=[
            pl.BlockSpec((BATCH, OBS_DIM), lambda j: (0, 0)),
            pl.BlockSpec((OBS_DIM, HIDDEN), lambda j: (0, 0)),
            pl.BlockSpec((1, HIDDEN), lambda j: (0, 0)),
            pl.BlockSpec((HIDDEN, BN), lambda j: (0, j)),
            pl.BlockSpec((1, BN), lambda j: (0, j)),
            pl.BlockSpec((BATCH, BN), lambda j: (0, j)),
        ],
        out_specs=[
            pl.BlockSpec((BATCH, BN), lambda j: (0, j)),
            pl.BlockSpec((BATCH, 16), lambda j: (0, 0)),
            pl.BlockSpec((BATCH, 16), lambda j: (0, 0)),
        ],
        out_shape=[
            jax.ShapeDtypeStruct((BATCH, NUM_ACTIONS), jnp.float32),
            jax.ShapeDtypeStruct((BATCH, 16), jnp.float32),
            jax.ShapeDtypeStruct((BATCH, 16), jnp.float32),
        ],
        scratch_shapes=[
            pltpu.VMEM((BATCH, HIDDEN), jnp.float32),
            pltpu.VMEM((BATCH, 1), jnp.float32),
            pltpu.VMEM((BATCH, 1), jnp.float32),
        ],
        compiler_params=pltpu.CompilerParams(
            dimension_semantics=("arbitrary",)),
    )(s, W1, b1r, W2, b2r, legal_actions)

    p = pl.pallas_call(
        _probs_kernel,
        grid=(NBLK,),
        in_specs=[
            pl.BlockSpec((BATCH, BN), lambda j: (0, j)),
            pl.BlockSpec((BATCH, 16), lambda j: (0, 0)),
            pl.BlockSpec((BATCH, 16), lambda j: (0, 0)),
        ],
        out_specs=pl.BlockSpec((BATCH, BN), lambda j: (0, j)),
        out_shape=jax.ShapeDtypeStruct((BATCH, NUM_ACTIONS), jnp.float32),
    )(key, mb, lsb)
    return p


CAP = 10240
L = 16
NW = 32
ROWS_PER_W = BATCH // NW


def _cmp_wins(ak, ai, bk, bi):
    return (ak > bk) | ((ak == bk) & (ai < bi))


def _exchange(lo_k, lo_i, hi_k, hi_i):
    rb_k = lax.rev(hi_k, (0,))
    rb_i = lax.rev(hi_i, (0,))
    w = _cmp_wins(lo_k, lo_i, rb_k, rb_i)
    top_k_ = jnp.where(w, lo_k, rb_k)
    top_i_ = jnp.where(w, lo_i, rb_i)
    bot_k_ = jnp.where(w, rb_k, lo_k)
    bot_i_ = jnp.where(w, rb_i, lo_i)
    bk2, bi2 = plsc.sort_key_val(bot_k_, bot_i_)
    tk2, ti2 = plsc.sort_key_val(top_k_, top_i_)
    return bk2, bi2, tk2, ti2


def _topk_sc_kernel(p_hbm, act_hbm, prob_hbm, row_a, row_b, cand_v, candi_v,
                    acto_v, probo_v, sem_a, sem_b):
    cid = lax.axis_index("c")
    sid = lax.axis_index("s")
    wid = sid * 2 + cid

    cp_a = pltpu.async_copy(p_hbm.at[wid * ROWS_PER_W], row_a, sem_a)
    cp_b = pltpu.async_copy(p_hbm.at[wid * ROWS_PER_W + 1], row_b, sem_b)

    lane = lax.iota(jnp.int32, L)

    for rr, (row_v, cp) in enumerate(((row_a, cp_a), (row_b, cp_b))):
        r = wid * ROWS_PER_W + rr
        cp.wait()

        def compact_body(i, off):
            v = row_v[pl.ds(i * L, L)]
            m = v > MIN_PROB
            cs = plsc.cumsum(m.astype(jnp.int32))
            pos = jnp.where(m, off + cs - 1, CAP + lane)
            plsc.store_scatter(cand_v, [pos], v)
            plsc.store_scatter(candi_v, [pos], lane + i * L)
            return off + cs[L - 1]

        off = lax.fori_loop(0, NUM_ACTIONS // L, compact_body, 0)

        cand_v[pl.ds(off, L)] = jnp.full((L,), -1.0, jnp.float32)
        candi_v[pl.ds(off, L)] = jnp.full((L,), -1, jnp.int32)
        nv = off // L + 1

        init = (jnp.full((L,), -1.0, jnp.float32), jnp.full((L,), -1, jnp.int32),
                jnp.full((L,), -1.0, jnp.float32), jnp.full((L,), -1, jnp.int32),
                jnp.full((L,), -1.0, jnp.float32), jnp.full((L,), -1, jnp.int32),
                jnp.full((L,), -1.0, jnp.float32), jnp.full((L,), -1, jnp.int32),
                jnp.float32(-1.0))

        def merge_body(k, carry):
            t0k, t0i, t1k, t1i, t2k, t2i, t3k, t3i, tmin = carry
            w = cand_v[pl.ds(k * L, L)]
            wi = candi_v[pl.ds(k * L, L)]
            wmax = jnp.max(w)

            def do_merge(_):
                ws, wis = plsc.sort_key_val(w, wi)
                rb_k = lax.rev(ws, (0,))
                rb_i = lax.rev(wis, (0,))
                win = _cmp_wins(t0k, t0i, rb_k, rb_i)
                n0k = jnp.where(win, t0k, rb_k)
                n0i = jnp.where(win, t0i, rb_i)
                n0k, n0i = plsc.sort_key_val(n0k, n0i)
                a0k, a0i, a1k, a1i = _exchange(n0k, n0i, t1k, t1i)
                a1k, a1i, a2k, a2i = _exchange(a1k, a1i, t2k, t2i)
                a2k, a2i, a3k, a3i = _exchange(a2k, a2i, t3k, t3i)
                return (a0k, a0i, a1k, a1i, a2k, a2i, a3k, a3i,
                        jnp.min(a0k))

            return lax.cond(wmax >= tmin, do_merge, lambda _: carry, None)

        t0k, t0i, t1k, t1i, t2k, t2i, t3k, t3i, _ = lax.fori_loop(
            0, nv, merge_body, init)

        for q, (tk_, ti_) in enumerate(
                ((t3k, t3i), (t2k, t2i), (t1k, t1i), (t0k, t0i))):
            m = tk_ > MIN_PROB
            a = jnp.where(m, ti_, -1)
            pv = jnp.where(m, tk_, 0.0)
            acto_v[pl.ds(q * L, L)] = lax.rev(a, (0,))
            probo_v[pl.ds(q * L, L)] = lax.rev(pv, (0,))

        pltpu.sync_copy(acto_v, act_hbm.at[r])
        pltpu.sync_copy(probo_v, prob_hbm.at[r])


def _topk_sc(p):
    mesh = plsc.VectorSubcoreMesh(core_axis_name="c", subcore_axis_name="s")
    f = pl.kernel(
        _topk_sc_kernel,
        out_type=(jax.ShapeDtypeStruct((BATCH, TOP_K), jnp.int32),
                  jax.ShapeDtypeStruct((BATCH, TOP_K), jnp.float32)),
        mesh=mesh,
        scratch_types=[
            pltpu.VMEM((NUM_ACTIONS,), jnp.float32),
            pltpu.VMEM((NUM_ACTIONS,), jnp.float32),
            pltpu.VMEM((CAP + L,), jnp.float32),
            pltpu.VMEM((CAP + L,), jnp.int32),
            pltpu.VMEM((TOP_K,), jnp.int32),
            pltpu.VMEM((TOP_K,), jnp.float32),
            pltpu.SemaphoreType.DMA,
            pltpu.SemaphoreType.DMA,
        ],
        compiler_params=pltpu.CompilerParams(needs_layout_passes=False),
    )
    return f(p)


def kernel(s, legal_actions, W1, b1, W2, b2, top_k):
    p = _mlp_probs(s, legal_actions, W1, b1, W2, b2)
    acts, probs = _topk_sc(p)
    valid = jnp.arange(TOP_K) < top_k
    top_k_actions = jnp.where(valid, acts, -1).astype(jnp.int32)
    top_k_probs = jnp.where(valid, probs, 0.0)
    return top_k_actions, top_k_probs


# speedup vs baseline: 3.2744x; 1.1376x over previous
import functools

import jax
import jax.numpy as jnp
from jax import lax
from jax.experimental import pallas as pl
from jax.experimental.pallas import tpu as pltpu
from jax.experimental.pallas import tpu_sc as plsc

OBS_DIM = 512
HIDDEN = 2048
NUM_ACTIONS = 32768
BATCH = 64
TOP_K = 64
MIN_PROB = 1e-4

BN = 2048
NBLK = NUM_ACTIONS // BN
NEG = -1e30


def _mlp_stats_kernel(s_ref, W1_ref, b1_ref, W2_ref, b2_ref, legal_ref,
                      key_ref, mb_ref, lsb_ref, h_ref, m_ref, ssum_ref):
    j = pl.program_id(0)

    @pl.when(j == 0)
    def _():
        h_ref[...] = jnp.tanh(
            jnp.dot(s_ref[...], W1_ref[...],
                    preferred_element_type=jnp.float32) + b1_ref[...])

    x = jnp.dot(h_ref[...], W2_ref[...],
                preferred_element_type=jnp.float32) + b2_ref[...]
    bmax = jnp.max(x, axis=1, keepdims=True)
    key_ref[...] = jnp.where(legal_ref[...] != 0.0, x, NEG)

    @pl.when(j == 0)
    def _():
        m_ref[...] = bmax
        ssum_ref[...] = jnp.sum(jnp.exp(x - bmax), axis=1, keepdims=True)

    @pl.when(j != 0)
    def _():
        m_old = m_ref[...]
        m_new = jnp.maximum(m_old, bmax)
        ssum_ref[...] = (ssum_ref[...] * jnp.exp(m_old - m_new)
                         + jnp.sum(jnp.exp(x - m_new), axis=1, keepdims=True))
        m_ref[...] = m_new

    @pl.when(j == NBLK - 1)
    def _():
        mb_ref[...] = jnp.broadcast_to(m_ref[...], (BATCH, 16))
        lsb_ref[...] = jnp.broadcast_to(jnp.log(ssum_ref[...]), (BATCH, 16))


def _probs_kernel(key_ref, mb_ref, lsb_ref, p_ref):
    t = (key_ref[...] - mb_ref[:, :1]) - lsb_ref[:, :1]
    p_ref[...] = jnp.exp(t)


def _mlp_probs(s, legal_actions, W1, b1, W2, b2):
    b1r = b1.reshape(1, HIDDEN)
    b2r = b2.reshape(1, NUM_ACTIONS)
    key, mb, lsb = pl.pallas_call(
        _mlp_stats_kernel,
        grid=(NBLK,),
        in_specs=[
            pl.BlockSpec((BATCH, OBS_DIM), lambda j: (0, 0)),
            pl.BlockSpec((OBS_DIM, HIDDEN), lambda j: (0, 0)),
            pl.BlockSpec((1, HIDDEN), lambda j: (0, 0)),
            pl.BlockSpec((HIDDEN, BN), lambda j: (0, j)),
            pl.BlockSpec((1, BN), lambda j: (0, j)),
            pl.BlockSpec((BATCH, BN), lambda j: (0, j)),
        ],
        out_specs=[
            pl.BlockSpec((BATCH, BN), lambda j: (0, j)),
            pl.BlockSpec((BATCH, 16), lambda j: (0, 0)),
            pl.BlockSpec((BATCH, 16), lambda j: (0, 0)),
        ],
        out_shape=[
            jax.ShapeDtypeStruct((BATCH, NUM_ACTIONS), jnp.float32),
            jax.ShapeDtypeStruct((BATCH, 16), jnp.float32),
            jax.ShapeDtypeStruct((BATCH, 16), jnp.float32),
        ],
        scratch_shapes=[
            pltpu.VMEM((BATCH, HIDDEN), jnp.float32),
            pltpu.VMEM((BATCH, 1), jnp.float32),
            pltpu.VMEM((BATCH, 1), jnp.float32),
        ],
        compiler_params=pltpu.CompilerParams(
            dimension_semantics=("arbitrary",)),
    )(s, W1, b1r, W2, b2r, legal_actions)

    p = pl.pallas_call(
        _probs_kernel,
        grid=(NBLK,),
        in_specs=[
            pl.BlockSpec((BATCH, BN), lambda j: (0, j)),
            pl.BlockSpec((BATCH, 16), lambda j: (0, 0)),
            pl.BlockSpec((BATCH, 16), lambda j: (0, 0)),
        ],
        out_specs=pl.BlockSpec((BATCH, BN), lambda j: (0, j)),
        out_shape=jax.ShapeDtypeStruct((BATCH, NUM_ACTIONS), jnp.float32),
    )(key, mb, lsb)
    return p


CAP = 10240
L = 16
NW = 32
ROWS_PER_W = BATCH // NW


def _cmp_wins(ak, ai, bk, bi):
    return (ak > bk) | ((ak == bk) & (ai < bi))


def _exchange(lo_k, lo_i, hi_k, hi_i):
    rb_k = lax.rev(hi_k, (0,))
    rb_i = lax.rev(hi_i, (0,))
    w = _cmp_wins(lo_k, lo_i, rb_k, rb_i)
    top_k_ = jnp.where(w, lo_k, rb_k)
    top_i_ = jnp.where(w, lo_i, rb_i)
    bot_k_ = jnp.where(w, rb_k, lo_k)
    bot_i_ = jnp.where(w, rb_i, lo_i)
    bk2, bi2 = plsc.sort_key_val(bot_k_, bot_i_)
    tk2, ti2 = plsc.sort_key_val(top_k_, top_i_)
    return bk2, bi2, tk2, ti2


def _topk_sc_kernel(p_hbm, act_hbm, prob_hbm, row_a, row_b, cand_v, candi_v,
                    acto_v, probo_v, sem_a, sem_b):
    cid = lax.axis_index("c")
    sid = lax.axis_index("s")
    wid = sid * 2 + cid

    cp_a = pltpu.async_copy(p_hbm.at[wid * ROWS_PER_W], row_a, sem_a)
    cp_b = pltpu.async_copy(p_hbm.at[wid * ROWS_PER_W + 1], row_b, sem_b)

    lane = lax.iota(jnp.int32, L)

    for rr, (row_v, cp) in enumerate(((row_a, cp_a), (row_b, cp_b))):
        r = wid * ROWS_PER_W + rr
        cp.wait()

        UNROLL = 4

        def compact_body(i, off):
            vs, css = [], []
            for u in range(UNROLL):
                v = row_v[pl.ds((i * UNROLL + u) * L, L)]
                vs.append(v)
                css.append(plsc.cumsum((v > MIN_PROB).astype(jnp.int32)))
            for u in range(UNROLL):
                v, cs = vs[u], css[u]
                pos = jnp.where(v > MIN_PROB, off + cs - 1, CAP + lane)
                plsc.store_scatter(cand_v, [pos], v)
                plsc.store_scatter(candi_v, [pos],
                                   lane + (i * UNROLL + u) * L)
                off = off + cs[L - 1]
            return off

        off = lax.fori_loop(0, NUM_ACTIONS // (L * UNROLL), compact_body, 0)

        cand_v[pl.ds(off, L)] = jnp.full((L,), -1.0, jnp.float32)
        candi_v[pl.ds(off, L)] = jnp.full((L,), -1, jnp.int32)
        nv = off // L + 1

        init = (jnp.full((L,), -1.0, jnp.float32), jnp.full((L,), -1, jnp.int32),
                jnp.full((L,), -1.0, jnp.float32), jnp.full((L,), -1, jnp.int32),
                jnp.full((L,), -1.0, jnp.float32), jnp.full((L,), -1, jnp.int32),
                jnp.full((L,), -1.0, jnp.float32), jnp.full((L,), -1, jnp.int32),
                jnp.float32(-1.0))

        def merge_body(k, carry):
            t0k, t0i, t1k, t1i, t2k, t2i, t3k, t3i, tmin = carry
            w = cand_v[pl.ds(k * L, L)]
            wi = candi_v[pl.ds(k * L, L)]
            wmax = jnp.max(w)

            def do_merge(_):
                ws, wis = plsc.sort_key_val(w, wi)
                rb_k = lax.rev(ws, (0,))
                rb_i = lax.rev(wis, (0,))
                win = _cmp_wins(t0k, t0i, rb_k, rb_i)
                n0k = jnp.where(win, t0k, rb_k)
                n0i = jnp.where(win, t0i, rb_i)
                n0k, n0i = plsc.sort_key_val(n0k, n0i)
                a0k, a0i, a1k, a1i = _exchange(n0k, n0i, t1k, t1i)
                a1k, a1i, a2k, a2i = _exchange(a1k, a1i, t2k, t2i)
                a2k, a2i, a3k, a3i = _exchange(a2k, a2i, t3k, t3i)
                return (a0k, a0i, a1k, a1i, a2k, a2i, a3k, a3i,
                        jnp.min(a0k))

            return lax.cond(wmax >= tmin, do_merge, lambda _: carry, None)

        t0k, t0i, t1k, t1i, t2k, t2i, t3k, t3i, _ = lax.fori_loop(
            0, nv, merge_body, init)

        for q, (tk_, ti_) in enumerate(
                ((t3k, t3i), (t2k, t2i), (t1k, t1i), (t0k, t0i))):
            m = tk_ > MIN_PROB
            a = jnp.where(m, ti_, -1)
            pv = jnp.where(m, tk_, 0.0)
            acto_v[pl.ds(q * L, L)] = lax.rev(a, (0,))
            probo_v[pl.ds(q * L, L)] = lax.rev(pv, (0,))

        pltpu.sync_copy(acto_v, act_hbm.at[r])
        pltpu.sync_copy(probo_v, prob_hbm.at[r])


def _topk_sc(p):
    mesh = plsc.VectorSubcoreMesh(core_axis_name="c", subcore_axis_name="s")
    f = pl.kernel(
        _topk_sc_kernel,
        out_type=(jax.ShapeDtypeStruct((BATCH, TOP_K), jnp.int32),
                  jax.ShapeDtypeStruct((BATCH, TOP_K), jnp.float32)),
        mesh=mesh,
        scratch_types=[
            pltpu.VMEM((NUM_ACTIONS,), jnp.float32),
            pltpu.VMEM((NUM_ACTIONS,), jnp.float32),
            pltpu.VMEM((CAP + L,), jnp.float32),
            pltpu.VMEM((CAP + L,), jnp.int32),
            pltpu.VMEM((TOP_K,), jnp.int32),
            pltpu.VMEM((TOP_K,), jnp.float32),
            pltpu.SemaphoreType.DMA,
            pltpu.SemaphoreType.DMA,
        ],
        compiler_params=pltpu.CompilerParams(needs_layout_passes=False),
    )
    return f(p)


def kernel(s, legal_actions, W1, b1, W2, b2, top_k):
    p = _mlp_probs(s, legal_actions, W1, b1, W2, b2)
    acts, probs = _topk_sc(p)
    valid = jnp.arange(TOP_K) < top_k
    top_k_actions = jnp.where(valid, acts, -1).astype(jnp.int32)
    top_k_probs = jnp.where(valid, probs, 0.0)
    return top_k_actions, top_k_probs


# speedup vs baseline: 3.5501x; 1.0842x over previous
import functools

import jax
import jax.numpy as jnp
from jax import lax
from jax.experimental import pallas as pl
from jax.experimental.pallas import tpu as pltpu
from jax.experimental.pallas import tpu_sc as plsc

OBS_DIM = 512
HIDDEN = 2048
NUM_ACTIONS = 32768
BATCH = 64
TOP_K = 64
MIN_PROB = 1e-4

BN = 2048
NBLK = NUM_ACTIONS // BN
NEG = -1e30


def _mlp_stats_kernel(s_ref, W1_ref, b1_ref, W2_ref, b2_ref, legal_ref,
                      key_ref, mb_ref, lsb_ref, h_ref, m_ref, ssum_ref):
    j = pl.program_id(0)

    @pl.when(j == 0)
    def _():
        h_ref[...] = jnp.tanh(
            jnp.dot(s_ref[...], W1_ref[...],
                    preferred_element_type=jnp.float32) + b1_ref[...])

    x = jnp.dot(h_ref[...], W2_ref[...],
                preferred_element_type=jnp.float32) + b2_ref[...]
    bmax = jnp.max(x, axis=1, keepdims=True)
    key_ref[...] = jnp.where(legal_ref[...] != 0.0, x, NEG)

    @pl.when(j == 0)
    def _():
        m_ref[...] = bmax
        ssum_ref[...] = jnp.sum(jnp.exp(x - bmax), axis=1, keepdims=True)

    @pl.when(j != 0)
    def _():
        m_old = m_ref[...]
        m_new = jnp.maximum(m_old, bmax)
        ssum_ref[...] = (ssum_ref[...] * jnp.exp(m_old - m_new)
                         + jnp.sum(jnp.exp(x - m_new), axis=1, keepdims=True))
        m_ref[...] = m_new

    @pl.when(j == NBLK - 1)
    def _():
        mb_ref[...] = jnp.broadcast_to(m_ref[...], (BATCH, 16))
        lsb_ref[...] = jnp.broadcast_to(jnp.log(ssum_ref[...]), (BATCH, 16))


def _probs_kernel(key_ref, mb_ref, lsb_ref, p_ref):
    t = (key_ref[...] - mb_ref[:, :1]) - lsb_ref[:, :1]
    p_ref[...] = jnp.exp(t)


def _mlp_probs(s, legal_actions, W1, b1, W2, b2):
    b1r = b1.reshape(1, HIDDEN)
    b2r = b2.reshape(1, NUM_ACTIONS)
    key, mb, lsb = pl.pallas_call(
        _mlp_stats_kernel,
        grid=(NBLK,),
        in_specs=[
            pl.BlockSpec((BATCH, OBS_DIM), lambda j: (0, 0)),
            pl.BlockSpec((OBS_DIM, HIDDEN), lambda j: (0, 0)),
            pl.BlockSpec((1, HIDDEN), lambda j: (0, 0)),
            pl.BlockSpec((HIDDEN, BN), lambda j: (0, j)),
            pl.BlockSpec((1, BN), lambda j: (0, j)),
            pl.BlockSpec((BATCH, BN), lambda j: (0, j)),
        ],
        out_specs=[
            pl.BlockSpec((BATCH, BN), lambda j: (0, j)),
            pl.BlockSpec((BATCH, 16), lambda j: (0, 0)),
            pl.BlockSpec((BATCH, 16), lambda j: (0, 0)),
        ],
        out_shape=[
            jax.ShapeDtypeStruct((BATCH, NUM_ACTIONS), jnp.float32),
            jax.ShapeDtypeStruct((BATCH, 16), jnp.float32),
            jax.ShapeDtypeStruct((BATCH, 16), jnp.float32),
        ],
        scratch_shapes=[
            pltpu.VMEM((BATCH, HIDDEN), jnp.float32),
            pltpu.VMEM((BATCH, 1), jnp.float32),
            pltpu.VMEM((BATCH, 1), jnp.float32),
        ],
        compiler_params=pltpu.CompilerParams(
            dimension_semantics=("arbitrary",)),
    )(s, W1, b1r, W2, b2r, legal_actions)

    p = pl.pallas_call(
        _probs_kernel,
        grid=(NBLK,),
        in_specs=[
            pl.BlockSpec((BATCH, BN), lambda j: (0, j)),
            pl.BlockSpec((BATCH, 16), lambda j: (0, 0)),
            pl.BlockSpec((BATCH, 16), lambda j: (0, 0)),
        ],
        out_specs=pl.BlockSpec((BATCH, BN), lambda j: (0, j)),
        out_shape=jax.ShapeDtypeStruct((BATCH, NUM_ACTIONS), jnp.float32),
    )(key, mb, lsb)
    return p


CAP = 10240
L = 16
NW = 32
ROWS_PER_W = BATCH // NW


def _cmp_wins(ak, ai, bk, bi):
    return (ak > bk) | ((ak == bk) & (ai < bi))


def _exchange(lo_k, lo_i, hi_k, hi_i):
    rb_k = lax.rev(hi_k, (0,))
    rb_i = lax.rev(hi_i, (0,))
    w = _cmp_wins(lo_k, lo_i, rb_k, rb_i)
    top_k_ = jnp.where(w, lo_k, rb_k)
    top_i_ = jnp.where(w, lo_i, rb_i)
    bot_k_ = jnp.where(w, rb_k, lo_k)
    bot_i_ = jnp.where(w, rb_i, lo_i)
    bk2, bi2 = plsc.sort_key_val(bot_k_, bot_i_)
    tk2, ti2 = plsc.sort_key_val(top_k_, top_i_)
    return bk2, bi2, tk2, ti2


def _topk_sc_kernel(p_hbm, act_hbm, prob_hbm, row_a, row_b, cand_v, candi_v,
                    acto_v, probo_v, sem_a, sem_b):
    cid = lax.axis_index("c")
    sid = lax.axis_index("s")
    wid = sid * 2 + cid

    cp_a = pltpu.async_copy(p_hbm.at[wid * ROWS_PER_W], row_a, sem_a)
    cp_b = pltpu.async_copy(p_hbm.at[wid * ROWS_PER_W + 1], row_b, sem_b)

    lane = lax.iota(jnp.int32, L)

    for rr, (row_v, cp) in enumerate(((row_a, cp_a), (row_b, cp_b))):
        r = wid * ROWS_PER_W + rr
        cp.wait()

        UNROLL = 8

        def compact_body(i, off):
            vs, css = [], []
            for u in range(UNROLL):
                v = row_v[pl.ds((i * UNROLL + u) * L, L)]
                vs.append(v)
                css.append(plsc.cumsum((v > MIN_PROB).astype(jnp.int32)))
            for u in range(UNROLL):
                v, cs = vs[u], css[u]
                pos = jnp.where(v > MIN_PROB, off + cs - 1, CAP + lane)
                plsc.store_scatter(cand_v, [pos], v)
                plsc.store_scatter(candi_v, [pos],
                                   lane + (i * UNROLL + u) * L)
                off = off + cs[L - 1]
            return off

        off = lax.fori_loop(0, NUM_ACTIONS // (L * UNROLL), compact_body, 0)

        cand_v[pl.ds(off, L)] = jnp.full((L,), -1.0, jnp.float32)
        candi_v[pl.ds(off, L)] = jnp.full((L,), -1, jnp.int32)
        nv = off // L + 1

        init = (jnp.full((L,), -1.0, jnp.float32), jnp.full((L,), -1, jnp.int32),
                jnp.full((L,), -1.0, jnp.float32), jnp.full((L,), -1, jnp.int32),
                jnp.full((L,), -1.0, jnp.float32), jnp.full((L,), -1, jnp.int32),
                jnp.full((L,), -1.0, jnp.float32), jnp.full((L,), -1, jnp.int32),
                jnp.float32(-1.0))

        def merge_body(k, carry):
            t0k, t0i, t1k, t1i, t2k, t2i, t3k, t3i, tmin = carry
            w = cand_v[pl.ds(k * L, L)]
            wi = candi_v[pl.ds(k * L, L)]
            wmax = jnp.max(w)

            def do_merge(_):
                ws, wis = plsc.sort_key_val(w, wi)
                rb_k = lax.rev(ws, (0,))
                rb_i = lax.rev(wis, (0,))
                win = _cmp_wins(t0k, t0i, rb_k, rb_i)
                n0k = jnp.where(win, t0k, rb_k)
                n0i = jnp.where(win, t0i, rb_i)
                n0k, n0i = plsc.sort_key_val(n0k, n0i)
                a0k, a0i, a1k, a1i = _exchange(n0k, n0i, t1k, t1i)
                a1k, a1i, a2k, a2i = _exchange(a1k, a1i, t2k, t2i)
                a2k, a2i, a3k, a3i = _exchange(a2k, a2i, t3k, t3i)
                return (a0k, a0i, a1k, a1i, a2k, a2i, a3k, a3i,
                        jnp.min(a0k))

            return lax.cond(wmax >= tmin, do_merge, lambda _: carry, None)

        t0k, t0i, t1k, t1i, t2k, t2i, t3k, t3i, _ = lax.fori_loop(
            0, nv, merge_body, init)

        for q, (tk_, ti_) in enumerate(
                ((t3k, t3i), (t2k, t2i), (t1k, t1i), (t0k, t0i))):
            m = tk_ > MIN_PROB
            a = jnp.where(m, ti_, -1)
            pv = jnp.where(m, tk_, 0.0)
            acto_v[pl.ds(q * L, L)] = lax.rev(a, (0,))
            probo_v[pl.ds(q * L, L)] = lax.rev(pv, (0,))

        pltpu.sync_copy(acto_v, act_hbm.at[r])
        pltpu.sync_copy(probo_v, prob_hbm.at[r])


def _topk_sc(p):
    mesh = plsc.VectorSubcoreMesh(core_axis_name="c", subcore_axis_name="s")
    f = pl.kernel(
        _topk_sc_kernel,
        out_type=(jax.ShapeDtypeStruct((BATCH, TOP_K), jnp.int32),
                  jax.ShapeDtypeStruct((BATCH, TOP_K), jnp.float32)),
        mesh=mesh,
        scratch_types=[
            pltpu.VMEM((NUM_ACTIONS,), jnp.float32),
            pltpu.VMEM((NUM_ACTIONS,), jnp.float32),
            pltpu.VMEM((CAP + L,), jnp.float32),
            pltpu.VMEM((CAP + L,), jnp.int32),
            pltpu.VMEM((TOP_K,), jnp.int32),
            pltpu.VMEM((TOP_K,), jnp.float32),
            pltpu.SemaphoreType.DMA,
            pltpu.SemaphoreType.DMA,
        ],
        compiler_params=pltpu.CompilerParams(needs_layout_passes=False),
    )
    return f(p)


def kernel(s, legal_actions, W1, b1, W2, b2, top_k):
    p = _mlp_probs(s, legal_actions, W1, b1, W2, b2)
    acts, probs = _topk_sc(p)
    valid = jnp.arange(TOP_K) < top_k
    top_k_actions = jnp.where(valid, acts, -1).astype(jnp.int32)
    top_k_probs = jnp.where(valid, probs, 0.0)
    return top_k_actions, top_k_probs


# speedup vs baseline: 3.6892x; 1.0392x over previous
import functools

import jax
import jax.numpy as jnp
from jax import lax
from jax.experimental import pallas as pl
from jax.experimental.pallas import tpu as pltpu
from jax.experimental.pallas import tpu_sc as plsc

OBS_DIM = 512
HIDDEN = 2048
NUM_ACTIONS = 32768
BATCH = 64
TOP_K = 64
MIN_PROB = 1e-4

BN = 2048
NBLK = NUM_ACTIONS // BN
NEG = -1e30


def _mlp_stats_kernel(s_ref, W1_ref, b1_ref, W2_ref, b2_ref, legal_ref,
                      key_ref, mb_ref, lsb_ref, h_ref, m_ref, ssum_ref):
    j = pl.program_id(0)

    @pl.when(j == 0)
    def _():
        h_ref[...] = jnp.tanh(
            jnp.dot(s_ref[...], W1_ref[...],
                    preferred_element_type=jnp.float32) + b1_ref[...])

    x = jnp.dot(h_ref[...], W2_ref[...],
                preferred_element_type=jnp.float32) + b2_ref[...]
    bmax = jnp.max(x, axis=1, keepdims=True)
    key_ref[...] = jnp.where(legal_ref[...] != 0.0, x, NEG)

    @pl.when(j == 0)
    def _():
        m_ref[...] = bmax
        ssum_ref[...] = jnp.sum(jnp.exp(x - bmax), axis=1, keepdims=True)

    @pl.when(j != 0)
    def _():
        m_old = m_ref[...]
        m_new = jnp.maximum(m_old, bmax)
        ssum_ref[...] = (ssum_ref[...] * jnp.exp(m_old - m_new)
                         + jnp.sum(jnp.exp(x - m_new), axis=1, keepdims=True))
        m_ref[...] = m_new

    @pl.when(j == NBLK - 1)
    def _():
        mb_ref[...] = jnp.broadcast_to(m_ref[...], (BATCH, 16))
        lsb_ref[...] = jnp.broadcast_to(jnp.log(ssum_ref[...]), (BATCH, 16))


def _probs_kernel(key_ref, mb_ref, lsb_ref, p_ref):
    t = (key_ref[...] - mb_ref[:, :1]) - lsb_ref[:, :1]
    p_ref[...] = jnp.exp(t)


def _mlp_probs(s, legal_actions, W1, b1, W2, b2):
    b1r = b1.reshape(1, HIDDEN)
    b2r = b2.reshape(1, NUM_ACTIONS)
    key, mb, lsb = pl.pallas_call(
        _mlp_stats_kernel,
        grid=(NBLK,),
        in_specs=[
            pl.BlockSpec((BATCH, OBS_DIM), lambda j: (0, 0)),
            pl.BlockSpec((OBS_DIM, HIDDEN), lambda j: (0, 0)),
            pl.BlockSpec((1, HIDDEN), lambda j: (0, 0)),
            pl.BlockSpec((HIDDEN, BN), lambda j: (0, j)),
            pl.BlockSpec((1, BN), lambda j: (0, j)),
            pl.BlockSpec((BATCH, BN), lambda j: (0, j)),
        ],
        out_specs=[
            pl.BlockSpec((BATCH, BN), lambda j: (0, j)),
            pl.BlockSpec((BATCH, 16), lambda j: (0, 0)),
            pl.BlockSpec((BATCH, 16), lambda j: (0, 0)),
        ],
        out_shape=[
            jax.ShapeDtypeStruct((BATCH, NUM_ACTIONS), jnp.float32),
            jax.ShapeDtypeStruct((BATCH, 16), jnp.float32),
            jax.ShapeDtypeStruct((BATCH, 16), jnp.float32),
        ],
        scratch_shapes=[
            pltpu.VMEM((BATCH, HIDDEN), jnp.float32),
            pltpu.VMEM((BATCH, 1), jnp.float32),
            pltpu.VMEM((BATCH, 1), jnp.float32),
        ],
        compiler_params=pltpu.CompilerParams(
            dimension_semantics=("arbitrary",)),
    )(s, W1, b1r, W2, b2r, legal_actions)

    p = pl.pallas_call(
        _probs_kernel,
        grid=(NBLK,),
        in_specs=[
            pl.BlockSpec((BATCH, BN), lambda j: (0, j)),
            pl.BlockSpec((BATCH, 16), lambda j: (0, 0)),
            pl.BlockSpec((BATCH, 16), lambda j: (0, 0)),
        ],
        out_specs=pl.BlockSpec((BATCH, BN), lambda j: (0, j)),
        out_shape=jax.ShapeDtypeStruct((BATCH, NUM_ACTIONS), jnp.float32),
    )(key, mb, lsb)
    return p


CAP = 10240
L = 16
NW = 32
ROWS_PER_W = BATCH // NW


def _cmp_wins(ak, ai, bk, bi):
    return (ak > bk) | ((ak == bk) & (ai < bi))


def _exchange(lo_k, lo_i, hi_k, hi_i):
    rb_k = lax.rev(hi_k, (0,))
    rb_i = lax.rev(hi_i, (0,))
    w = _cmp_wins(lo_k, lo_i, rb_k, rb_i)
    top_k_ = jnp.where(w, lo_k, rb_k)
    top_i_ = jnp.where(w, lo_i, rb_i)
    bot_k_ = jnp.where(w, rb_k, lo_k)
    bot_i_ = jnp.where(w, rb_i, lo_i)
    bk2, bi2 = plsc.sort_key_val(bot_k_, bot_i_)
    tk2, ti2 = plsc.sort_key_val(top_k_, top_i_)
    return bk2, bi2, tk2, ti2


def _topk_sc_kernel(p_hbm, act_hbm, prob_hbm, row_a, row_b, cand_v, candi_v,
                    acto_v, probo_v, sem_a, sem_b):
    cid = lax.axis_index("c")
    sid = lax.axis_index("s")
    wid = sid * 2 + cid

    cp_a = pltpu.async_copy(p_hbm.at[wid * ROWS_PER_W], row_a, sem_a)
    cp_b = pltpu.async_copy(p_hbm.at[wid * ROWS_PER_W + 1], row_b, sem_b)

    lane = lax.iota(jnp.int32, L)

    for rr, (row_v, cp) in enumerate(((row_a, cp_a), (row_b, cp_b))):
        r = wid * ROWS_PER_W + rr
        cp.wait()

        UNROLL = 16

        def compact_body(i, off):
            vs, css = [], []
            for u in range(UNROLL):
                v = row_v[pl.ds((i * UNROLL + u) * L, L)]
                vs.append(v)
                css.append(plsc.cumsum((v > MIN_PROB).astype(jnp.int32)))
            for u in range(UNROLL):
                v, cs = vs[u], css[u]
                pos = jnp.where(v > MIN_PROB, off + cs - 1, CAP + lane)
                plsc.store_scatter(cand_v, [pos], v)
                plsc.store_scatter(candi_v, [pos],
                                   lane + (i * UNROLL + u) * L)
                off = off + cs[L - 1]
            return off

        off = lax.fori_loop(0, NUM_ACTIONS // (L * UNROLL), compact_body, 0)

        cand_v[pl.ds(off, L)] = jnp.full((L,), -1.0, jnp.float32)
        candi_v[pl.ds(off, L)] = jnp.full((L,), -1, jnp.int32)
        nv = off // L + 1

        init = (jnp.full((L,), -1.0, jnp.float32), jnp.full((L,), -1, jnp.int32),
                jnp.full((L,), -1.0, jnp.float32), jnp.full((L,), -1, jnp.int32),
                jnp.full((L,), -1.0, jnp.float32), jnp.full((L,), -1, jnp.int32),
                jnp.full((L,), -1.0, jnp.float32), jnp.full((L,), -1, jnp.int32),
                jnp.float32(-1.0))

        def merge_body(k, carry):
            t0k, t0i, t1k, t1i, t2k, t2i, t3k, t3i, tmin = carry
            w = cand_v[pl.ds(k * L, L)]
            wi = candi_v[pl.ds(k * L, L)]
            wmax = jnp.max(w)

            def do_merge(_):
                ws, wis = plsc.sort_key_val(w, wi)
                rb_k = lax.rev(ws, (0,))
                rb_i = lax.rev(wis, (0,))
                win = _cmp_wins(t0k, t0i, rb_k, rb_i)
                n0k = jnp.where(win, t0k, rb_k)
                n0i = jnp.where(win, t0i, rb_i)
                n0k, n0i = plsc.sort_key_val(n0k, n0i)
                a0k, a0i, a1k, a1i = _exchange(n0k, n0i, t1k, t1i)
                a1k, a1i, a2k, a2i = _exchange(a1k, a1i, t2k, t2i)
                a2k, a2i, a3k, a3i = _exchange(a2k, a2i, t3k, t3i)
                return (a0k, a0i, a1k, a1i, a2k, a2i, a3k, a3i,
                        jnp.min(a0k))

            return lax.cond(wmax >= tmin, do_merge, lambda _: carry, None)

        t0k, t0i, t1k, t1i, t2k, t2i, t3k, t3i, _ = lax.fori_loop(
            0, nv, merge_body, init)

        for q, (tk_, ti_) in enumerate(
                ((t3k, t3i), (t2k, t2i), (t1k, t1i), (t0k, t0i))):
            m = tk_ > MIN_PROB
            a = jnp.where(m, ti_, -1)
            pv = jnp.where(m, tk_, 0.0)
            acto_v[pl.ds(q * L, L)] = lax.rev(a, (0,))
            probo_v[pl.ds(q * L, L)] = lax.rev(pv, (0,))

        pltpu.sync_copy(acto_v, act_hbm.at[r])
        pltpu.sync_copy(probo_v, prob_hbm.at[r])


def _topk_sc(p):
    mesh = plsc.VectorSubcoreMesh(core_axis_name="c", subcore_axis_name="s")
    f = pl.kernel(
        _topk_sc_kernel,
        out_type=(jax.ShapeDtypeStruct((BATCH, TOP_K), jnp.int32),
                  jax.ShapeDtypeStruct((BATCH, TOP_K), jnp.float32)),
        mesh=mesh,
        scratch_types=[
            pltpu.VMEM((NUM_ACTIONS,), jnp.float32),
            pltpu.VMEM((NUM_ACTIONS,), jnp.float32),
            pltpu.VMEM((CAP + L,), jnp.float32),
            pltpu.VMEM((CAP + L,), jnp.int32),
            pltpu.VMEM((TOP_K,), jnp.int32),
            pltpu.VMEM((TOP_K,), jnp.float32),
            pltpu.SemaphoreType.DMA,
            pltpu.SemaphoreType.DMA,
        ],
        compiler_params=pltpu.CompilerParams(needs_layout_passes=False),
    )
    return f(p)


def kernel(s, legal_actions, W1, b1, W2, b2, top_k):
    p = _mlp_probs(s, legal_actions, W1, b1, W2, b2)
    acts, probs = _topk_sc(p)
    valid = jnp.arange(TOP_K) < top_k
    top_k_actions = jnp.where(valid, acts, -1).astype(jnp.int32)
    top_k_probs = jnp.where(valid, probs, 0.0)
    return top_k_actions, top_k_probs


# speedup vs baseline: 3.7289x; 1.0108x over previous
import functools

import jax
import jax.numpy as jnp
from jax import lax
from jax.experimental import pallas as pl
from jax.experimental.pallas import tpu as pltpu
from jax.experimental.pallas import tpu_sc as plsc

OBS_DIM = 512
HIDDEN = 2048
NUM_ACTIONS = 32768
BATCH = 64
TOP_K = 64
MIN_PROB = 1e-4

BN = 2048
NBLK = NUM_ACTIONS // BN
NEG = -1e30


def _mlp_stats_kernel(s_ref, W1_ref, b1_ref, W2_ref, b2_ref, legal_ref,
                      key_ref, mb_ref, lsb_ref, h_ref, m_ref, ssum_ref):
    j = pl.program_id(0)

    @pl.when(j == 0)
    def _():
        h_ref[...] = jnp.tanh(
            jnp.dot(s_ref[...], W1_ref[...],
                    preferred_element_type=jnp.float32) + b1_ref[...])

    x = jnp.dot(h_ref[...], W2_ref[...],
                preferred_element_type=jnp.float32) + b2_ref[...]
    bmax = jnp.max(x, axis=1, keepdims=True)
    key_ref[...] = jnp.where(legal_ref[...] != 0.0, x, NEG)

    @pl.when(j == 0)
    def _():
        m_ref[...] = bmax
        ssum_ref[...] = jnp.sum(jnp.exp(x - bmax), axis=1, keepdims=True)

    @pl.when(j != 0)
    def _():
        m_old = m_ref[...]
        m_new = jnp.maximum(m_old, bmax)
        ssum_ref[...] = (ssum_ref[...] * jnp.exp(m_old - m_new)
                         + jnp.sum(jnp.exp(x - m_new), axis=1, keepdims=True))
        m_ref[...] = m_new

    @pl.when(j == NBLK - 1)
    def _():
        mb_ref[...] = jnp.broadcast_to(m_ref[...], (BATCH, 16))
        lsb_ref[...] = jnp.broadcast_to(jnp.log(ssum_ref[...]), (BATCH, 16))


def _probs_kernel(key_ref, mb_ref, lsb_ref, p_ref):
    t = (key_ref[...] - mb_ref[:, :1]) - lsb_ref[:, :1]
    p_ref[...] = jnp.exp(t)


def _mlp_probs(s, legal_actions, W1, b1, W2, b2):
    b1r = b1.reshape(1, HIDDEN)
    b2r = b2.reshape(1, NUM_ACTIONS)
    key, mb, lsb = pl.pallas_call(
        _mlp_stats_kernel,
        grid=(NBLK,),
        in_specs=[
            pl.BlockSpec((BATCH, OBS_DIM), lambda j: (0, 0)),
            pl.BlockSpec((OBS_DIM, HIDDEN), lambda j: (0, 0)),
            pl.BlockSpec((1, HIDDEN), lambda j: (0, 0)),
            pl.BlockSpec((HIDDEN, BN), lambda j: (0, j)),
            pl.BlockSpec((1, BN), lambda j: (0, j)),
            pl.BlockSpec((BATCH, BN), lambda j: (0, j)),
        ],
        out_specs=[
            pl.BlockSpec((BATCH, BN), lambda j: (0, j)),
            pl.BlockSpec((BATCH, 16), lambda j: (0, 0)),
            pl.BlockSpec((BATCH, 16), lambda j: (0, 0)),
        ],
        out_shape=[
            jax.ShapeDtypeStruct((BATCH, NUM_ACTIONS), jnp.float32),
            jax.ShapeDtypeStruct((BATCH, 16), jnp.float32),
            jax.ShapeDtypeStruct((BATCH, 16), jnp.float32),
        ],
        scratch_shapes=[
            pltpu.VMEM((BATCH, HIDDEN), jnp.float32),
            pltpu.VMEM((BATCH, 1), jnp.float32),
            pltpu.VMEM((BATCH, 1), jnp.float32),
        ],
        compiler_params=pltpu.CompilerParams(
            dimension_semantics=("arbitrary",)),
    )(s, W1, b1r, W2, b2r, legal_actions)

    p = pl.pallas_call(
        _probs_kernel,
        grid=(NBLK,),
        in_specs=[
            pl.BlockSpec((BATCH, BN), lambda j: (0, j)),
            pl.BlockSpec((BATCH, 16), lambda j: (0, 0)),
            pl.BlockSpec((BATCH, 16), lambda j: (0, 0)),
        ],
        out_specs=pl.BlockSpec((BATCH, BN), lambda j: (0, j)),
        out_shape=jax.ShapeDtypeStruct((BATCH, NUM_ACTIONS), jnp.float32),
    )(key, mb, lsb)
    return p


CAP = 10240
L = 16
NW = 32
ROWS_PER_W = BATCH // NW


def _cmp_wins(ak, ai, bk, bi):
    return (ak > bk) | ((ak == bk) & (ai < bi))


def _exchange(lo_k, lo_i, hi_k, hi_i):
    rb_k = lax.rev(hi_k, (0,))
    rb_i = lax.rev(hi_i, (0,))
    w = _cmp_wins(lo_k, lo_i, rb_k, rb_i)
    top_k_ = jnp.where(w, lo_k, rb_k)
    top_i_ = jnp.where(w, lo_i, rb_i)
    bot_k_ = jnp.where(w, rb_k, lo_k)
    bot_i_ = jnp.where(w, rb_i, lo_i)
    bk2, bi2 = plsc.sort_key_val(bot_k_, bot_i_)
    tk2, ti2 = plsc.sort_key_val(top_k_, top_i_)
    return bk2, bi2, tk2, ti2


def _topk_sc_kernel(p_hbm, act_hbm, prob_hbm, row_a, row_b, cand_v, candi_v,
                    acto_v, probo_v, sem_a, sem_b):
    cid = lax.axis_index("c")
    sid = lax.axis_index("s")
    wid = sid * 2 + cid

    cp_a = pltpu.async_copy(p_hbm.at[wid * ROWS_PER_W], row_a, sem_a)
    cp_b = pltpu.async_copy(p_hbm.at[wid * ROWS_PER_W + 1], row_b, sem_b)

    lane = lax.iota(jnp.int32, L)

    for rr, (row_v, cp) in enumerate(((row_a, cp_a), (row_b, cp_b))):
        r = wid * ROWS_PER_W + rr
        cp.wait()

        UNROLL = 32

        def compact_body(i, off):
            vs, css = [], []
            for u in range(UNROLL):
                v = row_v[pl.ds((i * UNROLL + u) * L, L)]
                vs.append(v)
                css.append(plsc.cumsum((v > MIN_PROB).astype(jnp.int32)))
            for u in range(UNROLL):
                v, cs = vs[u], css[u]
                pos = jnp.where(v > MIN_PROB, off + cs - 1, CAP + lane)
                plsc.store_scatter(cand_v, [pos], v)
                plsc.store_scatter(candi_v, [pos],
                                   lane + (i * UNROLL + u) * L)
                off = off + cs[L - 1]
            return off

        off = lax.fori_loop(0, NUM_ACTIONS // (L * UNROLL), compact_body, 0)

        cand_v[pl.ds(off, L)] = jnp.full((L,), -1.0, jnp.float32)
        candi_v[pl.ds(off, L)] = jnp.full((L,), -1, jnp.int32)
        nv = off // L + 1

        init = (jnp.full((L,), -1.0, jnp.float32), jnp.full((L,), -1, jnp.int32),
                jnp.full((L,), -1.0, jnp.float32), jnp.full((L,), -1, jnp.int32),
                jnp.full((L,), -1.0, jnp.float32), jnp.full((L,), -1, jnp.int32),
                jnp.full((L,), -1.0, jnp.float32), jnp.full((L,), -1, jnp.int32),
                jnp.float32(-1.0))

        def merge_body(k, carry):
            t0k, t0i, t1k, t1i, t2k, t2i, t3k, t3i, tmin = carry
            w = cand_v[pl.ds(k * L, L)]
            wi = candi_v[pl.ds(k * L, L)]
            wmax = jnp.max(w)

            def do_merge(_):
                ws, wis = plsc.sort_key_val(w, wi)
                rb_k = lax.rev(ws, (0,))
                rb_i = lax.rev(wis, (0,))
                win = _cmp_wins(t0k, t0i, rb_k, rb_i)
                n0k = jnp.where(win, t0k, rb_k)
                n0i = jnp.where(win, t0i, rb_i)
                n0k, n0i = plsc.sort_key_val(n0k, n0i)
                a0k, a0i, a1k, a1i = _exchange(n0k, n0i, t1k, t1i)
                a1k, a1i, a2k, a2i = _exchange(a1k, a1i, t2k, t2i)
                a2k, a2i, a3k, a3i = _exchange(a2k, a2i, t3k, t3i)
                return (a0k, a0i, a1k, a1i, a2k, a2i, a3k, a3i,
                        jnp.min(a0k))

            return lax.cond(wmax >= tmin, do_merge, lambda _: carry, None)

        t0k, t0i, t1k, t1i, t2k, t2i, t3k, t3i, _ = lax.fori_loop(
            0, nv, merge_body, init)

        for q, (tk_, ti_) in enumerate(
                ((t3k, t3i), (t2k, t2i), (t1k, t1i), (t0k, t0i))):
            m = tk_ > MIN_PROB
            a = jnp.where(m, ti_, -1)
            pv = jnp.where(m, tk_, 0.0)
            acto_v[pl.ds(q * L, L)] = lax.rev(a, (0,))
            probo_v[pl.ds(q * L, L)] = lax.rev(pv, (0,))

        pltpu.sync_copy(acto_v, act_hbm.at[r])
        pltpu.sync_copy(probo_v, prob_hbm.at[r])


def _topk_sc(p):
    mesh = plsc.VectorSubcoreMesh(core_axis_name="c", subcore_axis_name="s")
    f = pl.kernel(
        _topk_sc_kernel,
        out_type=(jax.ShapeDtypeStruct((BATCH, TOP_K), jnp.int32),
                  jax.ShapeDtypeStruct((BATCH, TOP_K), jnp.float32)),
        mesh=mesh,
        scratch_types=[
            pltpu.VMEM((NUM_ACTIONS,), jnp.float32),
            pltpu.VMEM((NUM_ACTIONS,), jnp.float32),
            pltpu.VMEM((CAP + L,), jnp.float32),
            pltpu.VMEM((CAP + L,), jnp.int32),
            pltpu.VMEM((TOP_K,), jnp.int32),
            pltpu.VMEM((TOP_K,), jnp.float32),
            pltpu.SemaphoreType.DMA,
            pltpu.SemaphoreType.DMA,
        ],
        compiler_params=pltpu.CompilerParams(needs_layout_passes=False),
    )
    return f(p)


def kernel(s, legal_actions, W1, b1, W2, b2, top_k):
    p = _mlp_probs(s, legal_actions, W1, b1, W2, b2)
    acts, probs = _topk_sc(p)
    valid = jnp.arange(TOP_K) < top_k
    top_k_actions = jnp.where(valid, acts, -1).astype(jnp.int32)
    top_k_probs = jnp.where(valid, probs, 0.0)
    return top_k_actions, top_k_probs
